# baseline (speedup 1.0000x reference)
import jax
import jax.numpy as jnp
from jax import lax
from jax.experimental import pallas as pl
from jax.experimental.pallas import tpu as pltpu
from jax.experimental.pallas import tpu_sc as plsc

R, C = 128, 32768
K = 16
NC, NS, L = 2, 16, 16
NW = NC * NS
ROWS_PER_W = R // NW
CHUNKS = C // L
U = 8
GL = U * L
NG = CHUNKS // U
BIG = 1e10


def _any_below(x, thr):
    pc = plsc.all_reduce_population_count(x < thr)
    return pc[0] > 0


def _group_min(buf, base):
    xs = [buf[pl.ds(base + u * L, L)] for u in range(U)]
    while len(xs) > 1:
        xs = [jnp.minimum(xs[i], xs[i + 1]) for i in range(0, len(xs), 2)]
    return xs[0]


def _merge_sorted(av, ai, bv, bi):
    rbv = lax.rev(bv, (0,))
    rbi = lax.rev(bi, (0,))
    take_a = av <= rbv
    lo = jnp.where(take_a, av, rbv)
    li = jnp.where(take_a, ai, rbi)
    return plsc.sort_key_val(lo, li)


def _group_tree(buf, base, lane):
    pairs = []
    for u in range(U):
        x = buf[pl.ds(base + u * L, L)]
        x = jnp.where(x != x, BIG, x)
        pairs.append(plsc.sort_key_val(x, lane + base + u * L))
    while len(pairs) > 1:
        pairs = [
            _merge_sorted(*pairs[i], *pairs[i + 1])
            for i in range(0, len(pairs), 2)
        ]
    return pairs[0]


def _scan_pair(bufA, bufB, slotA, slotB, sti, stv, thr_ref, lane):
    for buf, slot in ((bufA, slotA), (bufB, slotB)):
        gv, gi = _group_tree(buf, 0, lane)
        stv[slot] = gv
        sti[slot] = gi
        thr_ref[slot] = jnp.broadcast_to(gv[K - 1], (L,))

    def row_check(buf, slot, j):
        def upd(_):
            thr2 = thr_ref[slot]
            for u in range(U):
                x = buf[pl.ds(j * GL + u * L, L)]

                def mrg(_a, _x=x, _off=j * GL + u * L):
                    xx = jnp.where(_x != _x, BIG, _x)
                    xs, xi = plsc.sort_key_val(xx, lane + _off)
                    nbv, nbi = _merge_sorted(stv[slot], sti[slot], xs, xi)
                    stv[slot] = nbv
                    sti[slot] = nbi
                    thr_ref[slot] = jnp.broadcast_to(nbv[K - 1], (L,))
                    return _a

                thr2 = lax.cond(_any_below(x, thr2), mrg, lambda a: a, thr2)
                thr2 = thr_ref[slot]
            return 0

        m = _group_min(buf, j * GL)
        return lax.cond(_any_below(m, thr_ref[slot]), upd, lambda _: 0, 0)

    def step(j, carry):
        row_check(bufA, slotA, j)
        row_check(bufB, slotB, j)
        return carry

    lax.fori_loop(1, NG, step, 0)


def _sc_body(
    x_hbm, idx_hbm, val_hbm, buf0, buf1, buf2, sti, stv, thr_ref, s0, s1, s2
):
    wid = lax.axis_index("s") * NC + lax.axis_index("c")
    row0 = wid * ROWS_PER_W
    lane = lax.iota(jnp.int32, 16)

    cpA = pltpu.async_copy(x_hbm.at[row0], buf0, s0)
    cpB = pltpu.async_copy(x_hbm.at[row0 + 1], buf1, s1)
    cpA.wait()
    cpB.wait()
    cpC = pltpu.async_copy(x_hbm.at[row0 + 2], buf2, s2)
    _scan_pair(buf0, buf1, 0, 1, sti, stv, thr_ref, lane)

    cpD = pltpu.async_copy(x_hbm.at[row0 + 3], buf1, s1)
    cpC.wait()
    cpD.wait()
    _scan_pair(buf2, buf1, 2, 3, sti, stv, thr_ref, lane)

    pltpu.sync_copy(sti, idx_hbm.at[pl.ds(row0, ROWS_PER_W)])
    pltpu.sync_copy(stv, val_hbm.at[pl.ds(row0, ROWS_PER_W)])


@jax.jit
def _bottom_k(x):
    mesh = plsc.VectorSubcoreMesh(core_axis_name="c", subcore_axis_name="s")
    return pl.kernel(
        _sc_body,
        out_type=[
            jax.ShapeDtypeStruct((R, K), jnp.int32),
            jax.ShapeDtypeStruct((R, K), jnp.float32),
        ],
        mesh=mesh,
        compiler_params=pltpu.CompilerParams(needs_layout_passes=False),
        scratch_types=[
            pltpu.VMEM((C,), jnp.float32),
            pltpu.VMEM((C,), jnp.float32),
            pltpu.VMEM((C,), jnp.float32),
            pltpu.VMEM((ROWS_PER_W, K), jnp.int32),
            pltpu.VMEM((ROWS_PER_W, K), jnp.float32),
            pltpu.VMEM((ROWS_PER_W, K), jnp.float32),
            pltpu.SemaphoreType.DMA,
            pltpu.SemaphoreType.DMA,
            pltpu.SemaphoreType.DMA,
        ],
    )(x)


def kernel(dist_pot_donors, n_neighbors):
    idx, vals = _bottom_k(dist_pot_donors)
    idx = idx + (jnp.asarray(n_neighbors, dtype=idx.dtype) - K)
    return (idx, vals)


# speedup vs baseline: 1.7949x; 1.7949x over previous
import jax
import jax.numpy as jnp
from jax import lax
from jax.experimental import pallas as pl
from jax.experimental.pallas import tpu as pltpu
from jax.experimental.pallas import tpu_sc as plsc

R, C = 128, 32768
K = 16
NC, NS, L = 2, 16, 16
NW = NC * NS
ROWS_PER_W = R // NW
CHUNKS = C // L
U = 8
GL = U * L
NG = CHUNKS // U
BIG = 1e10


def _any_below(x, thr):
    pc = plsc.all_reduce_population_count(x < thr)
    return pc[0] > 0


def _group_min(buf, base):
    xs = [buf[pl.ds(base + u * L, L)] for u in range(U)]
    while len(xs) > 1:
        xs = [jnp.minimum(xs[i], xs[i + 1]) for i in range(0, len(xs), 2)]
    return xs[0]


def _merge_sorted(av, ai, bv, bi):
    rbv = lax.rev(bv, (0,))
    rbi = lax.rev(bi, (0,))
    take_a = av <= rbv
    lo = jnp.where(take_a, av, rbv)
    li = jnp.where(take_a, ai, rbi)
    return plsc.sort_key_val(lo, li)


def _group_tree(buf, base, lane):
    pairs = []
    for u in range(U):
        x = buf[pl.ds(base + u * L, L)]
        x = jnp.where(x != x, BIG, x)
        pairs.append(plsc.sort_key_val(x, lane + base + u * L))
    while len(pairs) > 1:
        pairs = [
            _merge_sorted(*pairs[i], *pairs[i + 1])
            for i in range(0, len(pairs), 2)
        ]
    return pairs[0]


def _pair_init(bufA, bufB, slotA, slotB, sti, stv, thr_ref, lane):
    for buf, slot in ((bufA, slotA), (bufB, slotB)):
        gv, gi = _group_tree(buf, 0, lane)
        stv[slot] = gv
        sti[slot] = gi
        thr_ref[slot] = jnp.broadcast_to(gv[K - 1], (L,))


def _scan_range(bufA, bufB, slotA, slotB, sti, stv, thr_ref, lane, j0, j1):
    def row_check(buf, slot, j):
        def upd(_):
            gv, gi = _group_tree(buf, j * GL, lane)
            nbv, nbi = _merge_sorted(stv[slot], sti[slot], gv, gi)
            stv[slot] = nbv
            sti[slot] = nbi
            thr_ref[slot] = jnp.broadcast_to(nbv[K - 1], (L,))
            return 0

        m = _group_min(buf, j * GL)
        return lax.cond(_any_below(m, thr_ref[slot]), upd, lambda _: 0, 0)

    def step(j, carry):
        row_check(bufA, slotA, j)
        row_check(bufB, slotB, j)
        return carry

    lax.fori_loop(j0, j1, step, 0)


H = C // 2
HG = NG // 2


def _sc_body(
    x_hbm, idx_hbm, val_hbm, buf0, buf1, buf2,
    sti, stv, thr_ref, s0, s1, s2, s3, s4
):
    wid = lax.axis_index("s") * NC + lax.axis_index("c")
    row0 = wid * ROWS_PER_W
    lane = lax.iota(jnp.int32, 16)
    lo = pl.ds(0, H)
    hi = pl.ds(H, H)

    cpA1 = pltpu.async_copy(x_hbm.at[row0, lo], buf0.at[lo], s0)
    cpB1 = pltpu.async_copy(x_hbm.at[row0 + 1, lo], buf1.at[lo], s1)
    cpA2 = pltpu.async_copy(x_hbm.at[row0, hi], buf0.at[hi], s2)
    cpB2 = pltpu.async_copy(x_hbm.at[row0 + 1, hi], buf1.at[hi], s3)
    cpA1.wait()
    cpB1.wait()
    cpC = pltpu.async_copy(x_hbm.at[row0 + 2], buf2, s4)
    _pair_init(buf0, buf1, 0, 1, sti, stv, thr_ref, lane)
    _scan_range(buf0, buf1, 0, 1, sti, stv, thr_ref, lane, 1, HG)
    cpA2.wait()
    cpB2.wait()
    cpD1 = pltpu.async_copy(x_hbm.at[row0 + 3, lo], buf1.at[lo], s0)
    _scan_range(buf0, buf1, 0, 1, sti, stv, thr_ref, lane, HG, NG)
    cpD2 = pltpu.async_copy(x_hbm.at[row0 + 3, hi], buf1.at[hi], s1)
    cpC.wait()
    cpD1.wait()
    _pair_init(buf2, buf1, 2, 3, sti, stv, thr_ref, lane)
    _scan_range(buf2, buf1, 2, 3, sti, stv, thr_ref, lane, 1, HG)
    cpD2.wait()
    _scan_range(buf2, buf1, 2, 3, sti, stv, thr_ref, lane, HG, NG)

    pltpu.sync_copy(sti, idx_hbm.at[pl.ds(row0, ROWS_PER_W)])
    pltpu.sync_copy(stv, val_hbm.at[pl.ds(row0, ROWS_PER_W)])


@jax.jit
def _bottom_k(x):
    mesh = plsc.VectorSubcoreMesh(core_axis_name="c", subcore_axis_name="s")
    return pl.kernel(
        _sc_body,
        out_type=[
            jax.ShapeDtypeStruct((R, K), jnp.int32),
            jax.ShapeDtypeStruct((R, K), jnp.float32),
        ],
        mesh=mesh,
        compiler_params=pltpu.CompilerParams(needs_layout_passes=False),
        scratch_types=[
            pltpu.VMEM((C,), jnp.float32),
            pltpu.VMEM((C,), jnp.float32),
            pltpu.VMEM((C,), jnp.float32),
            pltpu.VMEM((ROWS_PER_W, K), jnp.int32),
            pltpu.VMEM((ROWS_PER_W, K), jnp.float32),
            pltpu.VMEM((ROWS_PER_W, K), jnp.float32),
            pltpu.SemaphoreType.DMA,
            pltpu.SemaphoreType.DMA,
            pltpu.SemaphoreType.DMA,
            pltpu.SemaphoreType.DMA,
            pltpu.SemaphoreType.DMA,
        ],
    )(x)


def kernel(dist_pot_donors, n_neighbors):
    idx, vals = _bottom_k(dist_pot_donors)
    idx = idx + (jnp.asarray(n_neighbors, dtype=idx.dtype) - K)
    return (idx, vals)
